# Initial kernel scaffold; baseline (speedup 1.0000x reference)
import jax
import jax.numpy as jnp
from jax.experimental import pallas as pl


def kernel(x, k, local_idx):
    raise NotImplementedError("write your pallas kernel here")


# speedup vs baseline: 1.3860x; 1.3860x over previous
import functools

import jax
import jax.numpy as jnp
from jax import lax
from jax.experimental import pallas as pl

B, C, N = 8, 128, 1024
K = 20
K2 = 14
TN = 128
HIGHEST = lax.Precision.HIGHEST


def _knn_body(xt_tile_ref, xt_full_ref, x_full_ref, idx_ref, x1_ref):
    xt_tile = xt_tile_ref[0]
    xt_full = xt_full_ref[0]
    x_full = x_full_ref[0]

    xx = jnp.sum(x_full * x_full, axis=0, keepdims=True)
    dist = 2.0 * jnp.dot(xt_tile, x_full) - xx

    lane_iota = lax.broadcasted_iota(jnp.int32, (TN, N), 1)
    neg_inf = jnp.float32(-jnp.inf)

    idx_cols = []
    for _ in range(K):
        m = jnp.max(dist, axis=1, keepdims=True)
        amax = jnp.min(jnp.where(dist == m, lane_iota, N),
                       axis=1, keepdims=True)
        idx_cols.append(amax)
        dist = jnp.where(lane_iota == amax, neg_inf, dist)
    idx_tile = jnp.concatenate(idx_cols, axis=1)
    idx_ref[0] = idx_tile

    knn_parts = []
    s20 = jnp.zeros((TN, C), dtype=jnp.float32)
    for kk in range(K):
        oh = (lane_iota == idx_tile[:, kk:kk + 1]).astype(jnp.float32)
        g = jnp.dot(oh, xt_full, precision=HIGHEST)
        s20 = s20 + g
        knn_parts.append(g.reshape(TN, 1, C))
    knn = jnp.concatenate(knn_parts, axis=1)

    kk_iota = lax.broadcasted_iota(jnp.int32, (TN, K, C), 1)
    pos_inf = jnp.float32(jnp.inf)
    min_sum = jnp.zeros((TN, C), dtype=jnp.float32)
    for _ in range(K - K2):
        m = jnp.min(knn, axis=1, keepdims=True)
        amin = jnp.min(jnp.where(knn == m, kk_iota, K),
                       axis=1, keepdims=True)
        min_sum = min_sum + m[:, 0, :]
        knn = jnp.where(kk_iota == amin, pos_inf, knn)
    x1_ref[0] = (s20 - min_sum) * jnp.float32(1.0 / K2)


def _feature_body(idx_ref, x1_full_ref, x_tile_ref, out_ref):
    idx_tile = idx_ref[0]
    x1_full = x1_full_ref[0]
    x_tile = x_tile_ref[0]

    lane_iota = lax.broadcasted_iota(jnp.int32, (TN, N), 1)
    for kk in range(K):
        oh = (lane_iota == idx_tile[:, kk:kk + 1]).astype(jnp.float32)
        g_t = lax.dot_general(x1_full, oh, (((0,), (1,)), ((), ())),
                              precision=HIGHEST)
        out_ref[0, 0:C, :, kk] = g_t - x_tile
    out_ref[0, C:2 * C, :, :] = jnp.broadcast_to(
        x_tile.reshape(C, TN, 1), (C, TN, K))


@jax.jit
def _run(x):
    xt = jnp.transpose(x, (0, 2, 1))
    grid = (B, N // TN)
    idx, x1 = pl.pallas_call(
        _knn_body,
        grid=grid,
        in_specs=[
            pl.BlockSpec((1, TN, C), lambda b, i: (b, i, 0)),
            pl.BlockSpec((1, N, C), lambda b, i: (b, 0, 0)),
            pl.BlockSpec((1, C, N), lambda b, i: (b, 0, 0)),
        ],
        out_specs=[
            pl.BlockSpec((1, TN, K), lambda b, i: (b, i, 0)),
            pl.BlockSpec((1, TN, C), lambda b, i: (b, i, 0)),
        ],
        out_shape=[
            jax.ShapeDtypeStruct((B, N, K), jnp.int32),
            jax.ShapeDtypeStruct((B, N, C), jnp.float32),
        ],
    )(xt, xt, x)

    feature = pl.pallas_call(
        _feature_body,
        grid=grid,
        in_specs=[
            pl.BlockSpec((1, TN, K), lambda b, i: (b, i, 0)),
            pl.BlockSpec((1, N, C), lambda b, i: (b, 0, 0)),
            pl.BlockSpec((1, C, TN), lambda b, i: (b, 0, i)),
        ],
        out_specs=pl.BlockSpec((1, 2 * C, TN, K), lambda b, i: (b, 0, i, 0)),
        out_shape=jax.ShapeDtypeStruct((B, 2 * C, N, K), jnp.float32),
    )(idx, x1, x)
    return feature, idx


def kernel(x, k, local_idx):
    feature, idx = _run(x)
    idx = idx + (jnp.asarray(k, idx.dtype) - K)
    idx_base = jnp.arange(B, dtype=idx.dtype).reshape(-1, 1, 1) * N
    idx_flat = (idx + idx_base).reshape(-1)
    return feature, idx_flat


# speedup vs baseline: 4.9486x; 3.5705x over previous
import functools

import jax
import jax.numpy as jnp
from jax import lax
from jax.experimental import pallas as pl

B, C, N = 8, 128, 1024
K = 20
K2 = 14
TN = 128
HIGHEST = lax.Precision.HIGHEST


def _knn_body(xt_tile_ref, xt_full_ref, x_full_ref, idx_ref, x1_ref):
    xt_tile = xt_tile_ref[0]
    xt_full = xt_full_ref[0]
    x_full = x_full_ref[0]

    xx = jnp.sum(x_full * x_full, axis=0, keepdims=True)
    dist = 2.0 * jnp.dot(xt_tile, x_full) - xx

    lane_iota = lax.broadcasted_iota(jnp.int32, (TN, N), 1)
    neg_inf = jnp.float32(-jnp.inf)

    idx_cols = []
    for _ in range(K):
        m = jnp.max(dist, axis=1, keepdims=True)
        amax = jnp.min(jnp.where(dist == m, lane_iota, N),
                       axis=1, keepdims=True)
        idx_cols.append(amax)
        dist = jnp.where(lane_iota == amax, neg_inf, dist)
    idx_tile = jnp.concatenate(idx_cols, axis=1)
    idx_ref[0] = idx_tile

    knn_parts = []
    s20 = jnp.zeros((TN, C), dtype=jnp.float32)
    for kk in range(K):
        oh = (lane_iota == idx_tile[:, kk:kk + 1]).astype(jnp.float32)
        g = jnp.dot(oh, xt_full, precision=HIGHEST)
        s20 = s20 + g
        knn_parts.append(g.reshape(TN, 1, C))
    knn = jnp.concatenate(knn_parts, axis=1)

    kk_iota = lax.broadcasted_iota(jnp.int32, (TN, K, C), 1)
    pos_inf = jnp.float32(jnp.inf)
    min_sum = jnp.zeros((TN, C), dtype=jnp.float32)
    for _ in range(K - K2):
        m = jnp.min(knn, axis=1, keepdims=True)
        amin = jnp.min(jnp.where(knn == m, kk_iota, K),
                       axis=1, keepdims=True)
        min_sum = min_sum + m[:, 0, :]
        knn = jnp.where(kk_iota == amin, pos_inf, knn)
    x1_ref[0] = (s20 - min_sum) * jnp.float32(1.0 / K2)


def _feature_body(idx_ref, x1_full_ref, xt_tile_ref, out_ref):
    idx_tile = idx_ref[0]
    x1_full = x1_full_ref[0]
    xt_tile = xt_tile_ref[0]

    lane_iota = lax.broadcasted_iota(jnp.int32, (TN, N), 1)
    parts = []
    for kk in range(K):
        oh = (lane_iota == idx_tile[:, kk:kk + 1]).astype(jnp.float32)
        g = jnp.dot(oh, x1_full, precision=HIGHEST)
        parts.append(jnp.concatenate([g - xt_tile, xt_tile], axis=1))
    out_ref[0] = jnp.concatenate(parts, axis=1)


@jax.jit
def _run(x):
    xt = jnp.transpose(x, (0, 2, 1))
    grid = (B, N // TN)
    idx, x1 = pl.pallas_call(
        _knn_body,
        grid=grid,
        in_specs=[
            pl.BlockSpec((1, TN, C), lambda b, i: (b, i, 0)),
            pl.BlockSpec((1, N, C), lambda b, i: (b, 0, 0)),
            pl.BlockSpec((1, C, N), lambda b, i: (b, 0, 0)),
        ],
        out_specs=[
            pl.BlockSpec((1, TN, K), lambda b, i: (b, i, 0)),
            pl.BlockSpec((1, TN, C), lambda b, i: (b, i, 0)),
        ],
        out_shape=[
            jax.ShapeDtypeStruct((B, N, K), jnp.int32),
            jax.ShapeDtypeStruct((B, N, C), jnp.float32),
        ],
    )(xt, xt, x)

    f2 = pl.pallas_call(
        _feature_body,
        grid=grid,
        in_specs=[
            pl.BlockSpec((1, TN, K), lambda b, i: (b, i, 0)),
            pl.BlockSpec((1, N, C), lambda b, i: (b, 0, 0)),
            pl.BlockSpec((1, TN, C), lambda b, i: (b, i, 0)),
        ],
        out_specs=pl.BlockSpec((1, TN, K * 2 * C), lambda b, i: (b, i, 0)),
        out_shape=jax.ShapeDtypeStruct((B, N, K * 2 * C), jnp.float32),
    )(idx, x1, xt)
    feature = jnp.transpose(f2.reshape(B, N, K, 2 * C), (0, 3, 1, 2))
    return feature, idx


def kernel(x, k, local_idx):
    feature, idx = _run(x)
    idx = idx + (jnp.asarray(k, idx.dtype) - K)
    idx_base = jnp.arange(B, dtype=idx.dtype).reshape(-1, 1, 1) * N
    idx_flat = (idx + idx_base).reshape(-1)
    return feature, idx_flat
